# baseline (speedup 1.0000x reference)
import jax
import jax.numpy as jnp
from jax import lax
from jax.experimental import pallas as pl
from jax.experimental.pallas import tpu as pltpu
from jax.experimental.pallas import tpu_sc as plsc

END = 2
BATCH = 32
BEAM = 5
VOCAB = 100000
L = 16
NC, NS = 2, 16
BODY = 99968
SEGW = 512
CHUNK = 19968
NCHUNK = 5
SEGS_PER_CHUNK = CHUNK // SEGW
SHORT_OFF = NCHUNK * CHUNK
NSEG = 196
LAST_WIN = BODY - SEGW
NEG_INF = float("-inf")
BIG = 2**31 - 1

FETCH_PLAN = tuple((t * CHUNK, CHUNK) for t in range(NCHUNK)) + ((SHORT_OFF, 128),)


def _merge5(V, P, x, pos):
    for j in range(5):
        m = x > V[j]
        nv = jnp.where(m, x, V[j])
        np_ = jnp.where(m, pos, P[j])
        x = jnp.where(m, V[j], x)
        pos = jnp.where(m, P[j], pos)
        V[j], P[j] = nv, np_
    return V, P


def _argmax5(V, P):
    rv, rp = V[0], P[0]
    for j in range(1, 5):
        m = V[j] > rv
        rv = jnp.where(m, V[j], rv)
        rp = jnp.where(m, P[j], rp)
    v = jnp.max(rv)
    p = jnp.min(jnp.where(rv == v, rp, BIG))
    return v, p


def _sc_body(clp_hbm, tail_hbm, llp_hbm, lp_hbm,
             preds_hbm, vals_hbm, bp_hbm,
             buf0, buf1, buf2, union_v, llp_vm, lp_vm, tail_vm, idx_vm,
             cand_vals, cand_idx, o_preds, o_vals, o_bp,
             sem0, sem1, sem2, gsem, osem):
    w = lax.axis_index("s") * NC + lax.axis_index("c")
    lane = lax.iota(jnp.int32, L)
    minf = jnp.full((L,), NEG_INF, jnp.float32)
    zero_i = jnp.zeros((L,), jnp.int32)

    pltpu.sync_copy(llp_hbm.at[w], llp_vm)
    pltpu.sync_copy(lp_hbm.at[w], lp_vm)
    llp_vec = llp_vm[...]
    lp_vec = lp_vm[...]

    bufs = (buf0, buf1, buf2)
    sems = (sem0, sem1, sem2)

    def fetch(t, idx_slot):
        off, width = FETCH_PLAN[t]
        return pltpu.make_async_copy(
            clp_hbm.at[idx_slot, pl.ds(off, width)],
            bufs[t % 3].at[pl.ds(0, 1), pl.ds(0, width)],
            sems[t % 3])


    g0 = w * BEAM
    idx_vm[...] = jnp.full((L,), g0, jnp.int32)
    for t in range(3):
        fetch(t, idx_vm.at[pl.ds(0, 1)]).start()

    def row_body(r, _):
        g = w * BEAM + r
        cur_off = (r % 2) * 8
        nxt_off = 8 - cur_off
        idx_cur = idx_vm.at[pl.ds(cur_off, 1)]
        idx_nxt = idx_vm.at[pl.ds(nxt_off, 1)]
        idx_vm[...] = jnp.where(lane == nxt_off, g + 1, g)
        pltpu.sync_copy(tail_hbm.at[g], tail_vm)

        carry = (minf,) * 5 + (zero_i,) * 5
        for t in range(NCHUNK):
            fetch(t, idx_cur).wait()
            cur = bufs[t % 3]

            def seg_body(s3, kc, cur=cur, base=t * SEGS_PER_CHUNK):
                V = list(kc[0:5])
                P = list(kc[5:10])
                for u in range(3):
                    s = s3 * 3 + u
                    acc = cur[0, pl.ds(s * SEGW, L)]
                    for j in range(1, SEGW // L):
                        acc = jnp.maximum(
                            acc, cur[0, pl.ds(s * SEGW + j * L, L)])
                    pos = (base + s) * L + lane
                    V, P = _merge5(V, P, acc, pos)
                return tuple(V) + tuple(P)
            carry = lax.fori_loop(0, SEGS_PER_CHUNK // 3, seg_body, carry)

            if t + 3 <= NCHUNK:
                fetch(t + 3, idx_cur).start()
            else:
                @pl.when(r < BEAM - 1)
                def _():
                    fetch(t + 3 - 6, idx_nxt).start()

        fetch(5, idx_cur).wait()
        V = list(carry[0:5])
        P = list(carry[5:10])
        acc = bufs[2][0, pl.ds(0, L)]
        for j in range(1, 128 // L):
            acc = jnp.maximum(acc, bufs[2][0, pl.ds(j * L, L)])
        V, P = _merge5(V, P, acc, (NSEG - 1) * L + lane)

        @pl.when(r < BEAM - 1)
        def _():
            fetch(2, idx_nxt).start()

        found = []
        for _k in range(5):
            _, p = _argmax5(V, P)
            parent = p >> 4
            found.append(parent)
            for j in range(5):
                V[j] = jnp.where((P[j] >> 4) == parent, NEG_INF, V[j])

        offs = []
        for k in range(5):
            o = found[k] * SEGW
            o = jnp.where(o > LAST_WIN, LAST_WIN, o)
            offs.append(pl.multiple_of(o, 128))
        for k in range(5):
            pltpu.make_async_copy(
                clp_hbm.at[idx_cur, pl.ds(offs[k], SEGW)],
                union_v.at[pl.ds(0, 1), pl.ds(k * SEGW, SEGW)], gsem).start()
        for k in range(5):
            pltpu.make_async_copy(
                clp_hbm.at[idx_cur, pl.ds(offs[k], SEGW)],
                union_v.at[pl.ds(0, 1), pl.ds(k * SEGW, SEGW)], gsem).wait()

        V = [minf] * 5
        P = [zero_i] * 5
        for k in range(5):
            def win_body(i, kc, k=k, o=offs[k]):
                Vw = list(kc[0:5])
                Pw = list(kc[5:10])
                x = union_v[0, pl.ds(k * SEGW + i * L, L)]
                gid = o + i * L + lane
                Vw, Pw = _merge5(Vw, Pw, x, gid)
                return tuple(Vw) + tuple(Pw)
            kc = lax.fori_loop(0, SEGW // L, win_body, tuple(V) + tuple(P))
            V = list(kc[0:5])
            P = list(kc[5:10])
        V, P = _merge5(V, P, tail_vm[pl.ds(0, L)], BODY + lane)
        V, P = _merge5(V, P, tail_vm[pl.ds(L, L)], BODY + L + lane)

        cv = minf
        ci = zero_i
        for k in range(5):
            v, p = _argmax5(V, P)
            for j in range(5):
                V[j] = jnp.where(P[j] == p, NEG_INF, V[j])
            cv = jnp.where(lane == k, v, cv)
            ci = jnp.where(lane == k, p, ci)

        llp_r = jnp.max(jnp.where(lane == r, llp_vec, NEG_INF))
        pred_r = jnp.max(jnp.where(lane == r, lp_vec, jnp.int32(-1)))
        ended = pred_r == END
        end_vals = jnp.where(lane == 0, jnp.float32(0.0), minf)
        end_idx = jnp.where(lane == 0, 2,
                            jnp.where(lane == 1, 0,
                                      jnp.where(lane == 2, 1, lane)))
        cv = jnp.where(ended, end_vals, cv) + llp_r
        ci = jnp.where(ended, end_idx, ci)
        cand_vals[pl.ds(r * L, L)] = cv
        cand_idx[pl.ds(r * L, L)] = ci
        return 0

    lax.fori_loop(0, BEAM, row_body, 0)

    V = [jnp.full((L,), NEG_INF, jnp.float32) for _ in range(5)]
    P = [jnp.zeros((L,), jnp.int32) for _ in range(5)]
    for r in range(BEAM):
        x = cand_vals[pl.ds(r * L, L)]
        pos = r * L + lane
        V, P = _merge5(V, P, x, pos)

    ov = jnp.full((L,), NEG_INF, jnp.float32)
    oc = jnp.zeros((L,), jnp.int32)
    ob = jnp.zeros((L,), jnp.int32)
    for k in range(5):
        v, p = _argmax5(V, P)
        for j in range(5):
            V[j] = jnp.where(P[j] == p, NEG_INF, V[j])
        cls_vec = plsc.load_gather(cand_idx, [jnp.full((L,), p, jnp.int32)])
        ov = jnp.where(lane == k, v, ov)
        oc = jnp.where(lane == k, cls_vec, oc)
        ob = jnp.where(lane == k, p >> 4, ob)

    o_preds[...] = oc
    o_vals[...] = ov
    o_bp[...] = ob
    pltpu.make_async_copy(o_preds, preds_hbm.at[w], osem).start()
    pltpu.make_async_copy(o_vals, vals_hbm.at[w], osem).start()
    pltpu.make_async_copy(o_bp, bp_hbm.at[w], osem).start()
    pltpu.make_async_copy(o_preds, preds_hbm.at[w], osem).wait()
    pltpu.make_async_copy(o_vals, vals_hbm.at[w], osem).wait()
    pltpu.make_async_copy(o_bp, bp_hbm.at[w], osem).wait()


@jax.jit
def kernel(class_log_probs, last_log_probs, last_predictions):
    clp_tail = class_log_probs[:, BODY:]
    llp_pad = jnp.pad(last_log_probs, ((0, 0), (0, L - BEAM)))
    lp_pad = jnp.pad(last_predictions.reshape(BATCH, BEAM).astype(jnp.int32),
                     ((0, 0), (0, L - BEAM)))

    mesh = plsc.VectorSubcoreMesh(core_axis_name="c", subcore_axis_name="s",
                                  num_cores=NC, num_subcores=NS)
    sc_call = pl.kernel(
        _sc_body,
        out_type=[
            jax.ShapeDtypeStruct((BATCH, L), jnp.int32),
            jax.ShapeDtypeStruct((BATCH, L), jnp.float32),
            jax.ShapeDtypeStruct((BATCH, L), jnp.int32),
        ],
        mesh=mesh,
        compiler_params=pltpu.CompilerParams(needs_layout_passes=False),
        scratch_types=[
            pltpu.VMEM((1, CHUNK), jnp.float32),
            pltpu.VMEM((1, CHUNK), jnp.float32),
            pltpu.VMEM((1, CHUNK), jnp.float32),
            pltpu.VMEM((1, 5 * SEGW), jnp.float32),
            pltpu.VMEM((L,), jnp.float32),
            pltpu.VMEM((L,), jnp.int32),
            pltpu.VMEM((VOCAB - BODY,), jnp.float32),
            pltpu.VMEM((L,), jnp.int32),
            pltpu.VMEM((BEAM * L,), jnp.float32),
            pltpu.VMEM((BEAM * L,), jnp.int32),
            pltpu.VMEM((L,), jnp.int32),
            pltpu.VMEM((L,), jnp.float32),
            pltpu.VMEM((L,), jnp.int32),
            pltpu.SemaphoreType.DMA,
            pltpu.SemaphoreType.DMA,
            pltpu.SemaphoreType.DMA,
            pltpu.SemaphoreType.DMA,
            pltpu.SemaphoreType.DMA,
        ],
    )
    preds16, vals16, bp16 = sc_call(class_log_probs, clp_tail, llp_pad, lp_pad)
    return (preds16[:, :BEAM], vals16[:, :BEAM], bp16[:, :BEAM])


# speedup vs baseline: 1.0795x; 1.0795x over previous
import jax
import jax.numpy as jnp
from jax import lax
from jax.experimental import pallas as pl
from jax.experimental.pallas import tpu as pltpu
from jax.experimental.pallas import tpu_sc as plsc

END = 2
BATCH = 32
BEAM = 5
VOCAB = 100000
L = 16
NC, NS = 2, 16
BODY = 99968
SEGW = 512
CHUNK = 19968
NCHUNK = 5
SEGS_PER_CHUNK = CHUNK // SEGW
SHORT_OFF = NCHUNK * CHUNK
NSEG = 196
LAST_WIN = BODY - SEGW
NEG_INF = float("-inf")
BIG = 2**31 - 1

FETCH_PLAN = tuple((t * CHUNK, CHUNK) for t in range(NCHUNK)) + ((SHORT_OFF, 128),)


def _merge5(V, P, x, pos):
    for j in range(5):
        m = x > V[j]
        nv = jnp.where(m, x, V[j])
        np_ = jnp.where(m, pos, P[j])
        x = jnp.where(m, V[j], x)
        pos = jnp.where(m, P[j], pos)
        V[j], P[j] = nv, np_
    return V, P


def _argmax5(V, P):
    rv, rp = V[0], P[0]
    for j in range(1, 5):
        m = V[j] > rv
        rv = jnp.where(m, V[j], rv)
        rp = jnp.where(m, P[j], rp)
    v = jnp.max(rv)
    p = jnp.min(jnp.where(rv == v, rp, BIG))
    return v, p


def _sc_body(clp_hbm, tail_hbm, llp_hbm, lp_hbm,
             preds_hbm, vals_hbm, bp_hbm,
             buf0, buf1, buf2, union_v, llp_vm, lp_vm, tail_vm, idx_vm,
             cand_vals, cand_idx, o_preds, o_vals, o_bp,
             sem0, sem1, sem2, gsem, osem):
    w = lax.axis_index("s") * NC + lax.axis_index("c")
    lane = lax.iota(jnp.int32, L)
    minf = jnp.full((L,), NEG_INF, jnp.float32)
    zero_i = jnp.zeros((L,), jnp.int32)

    pltpu.sync_copy(llp_hbm.at[w], llp_vm)
    pltpu.sync_copy(lp_hbm.at[w], lp_vm)
    llp_vec = llp_vm[...]
    lp_vec = lp_vm[...]

    bufs = (buf0, buf1, buf2)
    sems = (sem0, sem1, sem2)

    def fetch(t, idx_slot):
        off, width = FETCH_PLAN[t]
        return pltpu.make_async_copy(
            clp_hbm.at[idx_slot, pl.ds(off, width)],
            bufs[t % 3].at[pl.ds(0, 1), pl.ds(0, width)],
            sems[t % 3])


    g0 = w * BEAM
    idx_vm[...] = jnp.full((L,), g0, jnp.int32)
    for t in range(3):
        fetch(t, idx_vm.at[pl.ds(0, 1)]).start()

    def row_body(r, _):
        g = w * BEAM + r
        cur_off = (r % 2) * 8
        nxt_off = 8 - cur_off
        idx_cur = idx_vm.at[pl.ds(cur_off, 1)]
        idx_nxt = idx_vm.at[pl.ds(nxt_off, 1)]
        idx_vm[...] = jnp.where(lane == nxt_off, g + 1, g)
        pltpu.sync_copy(tail_hbm.at[g], tail_vm)

        carry = (minf,) * 5 + (zero_i,) * 5
        for t in range(NCHUNK):
            fetch(t, idx_cur).wait()
            cur = bufs[t % 3]

            def seg_body(s, kc, cur=cur, base=t * SEGS_PER_CHUNK):
                V = list(kc[0:5])
                P = list(kc[5:10])
                acc = cur[0, pl.ds(s * SEGW, L)]
                for j in range(1, SEGW // L):
                    acc = jnp.maximum(acc, cur[0, pl.ds(s * SEGW + j * L, L)])
                pos = (base + s) * L + lane
                V, P = _merge5(V, P, acc, pos)
                return tuple(V) + tuple(P)
            carry = lax.fori_loop(0, SEGS_PER_CHUNK, seg_body, carry)

            if t + 3 <= NCHUNK:
                fetch(t + 3, idx_cur).start()
            else:
                @pl.when(r < BEAM - 1)
                def _():
                    fetch(t + 3 - 6, idx_nxt).start()

        fetch(5, idx_cur).wait()
        V = list(carry[0:5])
        P = list(carry[5:10])
        acc = bufs[2][0, pl.ds(0, L)]
        for j in range(1, 128 // L):
            acc = jnp.maximum(acc, bufs[2][0, pl.ds(j * L, L)])
        V, P = _merge5(V, P, acc, (NSEG - 1) * L + lane)

        @pl.when(r < BEAM - 1)
        def _():
            fetch(2, idx_nxt).start()

        found = []
        for _k in range(5):
            _, p = _argmax5(V, P)
            parent = p >> 4
            found.append(parent)
            for j in range(5):
                V[j] = jnp.where((P[j] >> 4) == parent, NEG_INF, V[j])

        offs = []
        for k in range(5):
            o = found[k] * SEGW
            o = jnp.where(o > LAST_WIN, LAST_WIN, o)
            offs.append(pl.multiple_of(o, 128))
        for k in range(5):
            pltpu.make_async_copy(
                clp_hbm.at[idx_cur, pl.ds(offs[k], SEGW)],
                union_v.at[pl.ds(0, 1), pl.ds(k * SEGW, SEGW)], gsem).start()
        for k in range(5):
            pltpu.make_async_copy(
                clp_hbm.at[idx_cur, pl.ds(offs[k], SEGW)],
                union_v.at[pl.ds(0, 1), pl.ds(k * SEGW, SEGW)], gsem).wait()

        V = [minf] * 5
        P = [zero_i] * 5
        for k in range(5):
            def win_body(i, kc, k=k, o=offs[k]):
                Vw = list(kc[0:5])
                Pw = list(kc[5:10])
                x = union_v[0, pl.ds(k * SEGW + i * L, L)]
                gid = o + i * L + lane
                Vw, Pw = _merge5(Vw, Pw, x, gid)
                return tuple(Vw) + tuple(Pw)
            kc = lax.fori_loop(0, SEGW // L, win_body, tuple(V) + tuple(P))
            V = list(kc[0:5])
            P = list(kc[5:10])
        V, P = _merge5(V, P, tail_vm[pl.ds(0, L)], BODY + lane)
        V, P = _merge5(V, P, tail_vm[pl.ds(L, L)], BODY + L + lane)

        cv = minf
        ci = zero_i
        for k in range(5):
            v, p = _argmax5(V, P)
            for j in range(5):
                V[j] = jnp.where(P[j] == p, NEG_INF, V[j])
            cv = jnp.where(lane == k, v, cv)
            ci = jnp.where(lane == k, p, ci)

        llp_r = jnp.max(jnp.where(lane == r, llp_vec, NEG_INF))
        pred_r = jnp.max(jnp.where(lane == r, lp_vec, jnp.int32(-1)))
        ended = pred_r == END
        end_vals = jnp.where(lane == 0, jnp.float32(0.0), minf)
        end_idx = jnp.where(lane == 0, 2,
                            jnp.where(lane == 1, 0,
                                      jnp.where(lane == 2, 1, lane)))
        cv = jnp.where(ended, end_vals, cv) + llp_r
        ci = jnp.where(ended, end_idx, ci)
        cand_vals[pl.ds(r * L, L)] = cv
        cand_idx[pl.ds(r * L, L)] = ci
        return 0

    lax.fori_loop(0, BEAM, row_body, 0)

    V = [jnp.full((L,), NEG_INF, jnp.float32) for _ in range(5)]
    P = [jnp.zeros((L,), jnp.int32) for _ in range(5)]
    for r in range(BEAM):
        x = cand_vals[pl.ds(r * L, L)]
        pos = r * L + lane
        V, P = _merge5(V, P, x, pos)

    ov = jnp.full((L,), NEG_INF, jnp.float32)
    oc = jnp.zeros((L,), jnp.int32)
    ob = jnp.zeros((L,), jnp.int32)
    for k in range(5):
        v, p = _argmax5(V, P)
        for j in range(5):
            V[j] = jnp.where(P[j] == p, NEG_INF, V[j])
        cls_vec = plsc.load_gather(cand_idx, [jnp.full((L,), p, jnp.int32)])
        ov = jnp.where(lane == k, v, ov)
        oc = jnp.where(lane == k, cls_vec, oc)
        ob = jnp.where(lane == k, p >> 4, ob)

    o_preds[...] = oc
    o_vals[...] = ov
    o_bp[...] = ob
    pltpu.make_async_copy(o_preds, preds_hbm.at[w], osem).start()
    pltpu.make_async_copy(o_vals, vals_hbm.at[w], osem).start()
    pltpu.make_async_copy(o_bp, bp_hbm.at[w], osem).start()
    pltpu.make_async_copy(o_preds, preds_hbm.at[w], osem).wait()
    pltpu.make_async_copy(o_vals, vals_hbm.at[w], osem).wait()
    pltpu.make_async_copy(o_bp, bp_hbm.at[w], osem).wait()


@jax.jit
def kernel(class_log_probs, last_log_probs, last_predictions):
    clp_tail = class_log_probs[:, BODY:]
    llp_pad = jnp.pad(last_log_probs, ((0, 0), (0, L - BEAM)))
    lp_pad = jnp.pad(last_predictions.reshape(BATCH, BEAM).astype(jnp.int32),
                     ((0, 0), (0, L - BEAM)))

    mesh = plsc.VectorSubcoreMesh(core_axis_name="c", subcore_axis_name="s",
                                  num_cores=NC, num_subcores=NS)
    sc_call = pl.kernel(
        _sc_body,
        out_type=[
            jax.ShapeDtypeStruct((BATCH, L), jnp.int32),
            jax.ShapeDtypeStruct((BATCH, L), jnp.float32),
            jax.ShapeDtypeStruct((BATCH, L), jnp.int32),
        ],
        mesh=mesh,
        compiler_params=pltpu.CompilerParams(needs_layout_passes=False),
        scratch_types=[
            pltpu.VMEM((1, CHUNK), jnp.float32),
            pltpu.VMEM((1, CHUNK), jnp.float32),
            pltpu.VMEM((1, CHUNK), jnp.float32),
            pltpu.VMEM((1, 5 * SEGW), jnp.float32),
            pltpu.VMEM((L,), jnp.float32),
            pltpu.VMEM((L,), jnp.int32),
            pltpu.VMEM((VOCAB - BODY,), jnp.float32),
            pltpu.VMEM((L,), jnp.int32),
            pltpu.VMEM((BEAM * L,), jnp.float32),
            pltpu.VMEM((BEAM * L,), jnp.int32),
            pltpu.VMEM((L,), jnp.int32),
            pltpu.VMEM((L,), jnp.float32),
            pltpu.VMEM((L,), jnp.int32),
            pltpu.SemaphoreType.DMA,
            pltpu.SemaphoreType.DMA,
            pltpu.SemaphoreType.DMA,
            pltpu.SemaphoreType.DMA,
            pltpu.SemaphoreType.DMA,
        ],
    )
    preds16, vals16, bp16 = sc_call(class_log_probs, clp_tail, llp_pad, lp_pad)
    return (preds16[:, :BEAM], vals16[:, :BEAM], bp16[:, :BEAM])


# speedup vs baseline: 1.1112x; 1.0294x over previous
import jax
import jax.numpy as jnp
from jax import lax
from jax.experimental import pallas as pl
from jax.experimental.pallas import tpu as pltpu
from jax.experimental.pallas import tpu_sc as plsc

END = 2
BATCH = 32
BEAM = 5
VOCAB = 100000
L = 16
NC, NS = 2, 16
BODY = 99968
SEGW = 512
CHUNK = 19968
NCHUNK = 5
SEGS_PER_CHUNK = CHUNK // SEGW
SHORT_OFF = NCHUNK * CHUNK
NSEG = 196
LAST_WIN = BODY - SEGW
NEG_INF = float("-inf")
BIG = 2**31 - 1

FETCH_PLAN = tuple((t * CHUNK, CHUNK) for t in range(NCHUNK)) + ((SHORT_OFF, 128),)


def _merge5(V, P, x, pos):
    for j in range(5):
        m = x > V[j]
        nv = jnp.where(m, x, V[j])
        np_ = jnp.where(m, pos, P[j])
        x = jnp.where(m, V[j], x)
        pos = jnp.where(m, P[j], pos)
        V[j], P[j] = nv, np_
    return V, P


def _argmax5(V, P):
    rv, rp = V[0], P[0]
    for j in range(1, 5):
        m = V[j] > rv
        rv = jnp.where(m, V[j], rv)
        rp = jnp.where(m, P[j], rp)
    v = jnp.max(rv)
    p = jnp.min(jnp.where(rv == v, rp, BIG))
    return v, p


def _sc_body(clp_hbm, aux_hbm,
             out_hbm,
             buf0, buf1, buf2, union_v, aux_vm, idx_vm,
             cand_vals, cand_idx, o_all,
             sem0, sem1, sem2, gsem, osem):
    w = lax.axis_index("s") * NC + lax.axis_index("c")
    lane = lax.iota(jnp.int32, L)
    minf = jnp.full((L,), NEG_INF, jnp.float32)
    zero_i = jnp.zeros((L,), jnp.int32)

    bufs = (buf0, buf1, buf2)
    sems = (sem0, sem1, sem2)

    def fetch(t, idx_slot):
        off, width = FETCH_PLAN[t]
        return pltpu.make_async_copy(
            clp_hbm.at[idx_slot, pl.ds(off, width)],
            bufs[t % 3].at[pl.ds(0, 1), pl.ds(0, width)],
            sems[t % 3])


    g0 = w * BEAM
    idx_vm[...] = jnp.full((L,), g0, jnp.int32)
    for t in range(3):
        fetch(t, idx_vm.at[pl.ds(0, 1)]).start()

    def row_body(r, _):
        g = w * BEAM + r
        cur_off = (r % 2) * 8
        nxt_off = 8 - cur_off
        idx_cur = idx_vm.at[pl.ds(cur_off, 1)]
        idx_nxt = idx_vm.at[pl.ds(nxt_off, 1)]
        idx_vm[...] = jnp.where(lane == nxt_off, g + 1, g)
        pltpu.sync_copy(aux_hbm.at[g], aux_vm)

        carry = (minf,) * 5 + (zero_i,) * 5
        for t in range(NCHUNK):
            fetch(t, idx_cur).wait()
            cur = bufs[t % 3]

            def seg_body(s, kc, cur=cur, base=t * SEGS_PER_CHUNK):
                V = list(kc[0:5])
                P = list(kc[5:10])
                acc = cur[0, pl.ds(s * SEGW, L)]
                for j in range(1, SEGW // L):
                    acc = jnp.maximum(acc, cur[0, pl.ds(s * SEGW + j * L, L)])
                pos = (base + s) * L + lane
                V, P = _merge5(V, P, acc, pos)
                return tuple(V) + tuple(P)
            carry = lax.fori_loop(0, SEGS_PER_CHUNK, seg_body, carry)

            if t + 3 <= NCHUNK:
                fetch(t + 3, idx_cur).start()
            else:
                @pl.when(r < BEAM - 1)
                def _():
                    fetch(t + 3 - 6, idx_nxt).start()

        fetch(5, idx_cur).wait()
        V = list(carry[0:5])
        P = list(carry[5:10])
        acc = bufs[2][0, pl.ds(0, L)]
        for j in range(1, 128 // L):
            acc = jnp.maximum(acc, bufs[2][0, pl.ds(j * L, L)])
        V, P = _merge5(V, P, acc, (NSEG - 1) * L + lane)

        @pl.when(r < BEAM - 1)
        def _():
            fetch(2, idx_nxt).start()

        found = []
        for _k in range(5):
            _, p = _argmax5(V, P)
            parent = p >> 4
            found.append(parent)
            for j in range(5):
                V[j] = jnp.where((P[j] >> 4) == parent, NEG_INF, V[j])

        offs = []
        for k in range(5):
            o = found[k] * SEGW
            o = jnp.where(o > LAST_WIN, LAST_WIN, o)
            offs.append(pl.multiple_of(o, 128))
        for k in range(5):
            pltpu.make_async_copy(
                clp_hbm.at[idx_cur, pl.ds(offs[k], SEGW)],
                union_v.at[pl.ds(0, 1), pl.ds(k * SEGW, SEGW)], gsem).start()
        for k in range(5):
            pltpu.make_async_copy(
                clp_hbm.at[idx_cur, pl.ds(offs[k], SEGW)],
                union_v.at[pl.ds(0, 1), pl.ds(k * SEGW, SEGW)], gsem).wait()

        V = [minf] * 5
        P = [zero_i] * 5
        for k in range(5):
            def win_body(i, kc, k=k, o=offs[k]):
                Vw = list(kc[0:5])
                Pw = list(kc[5:10])
                x = union_v[0, pl.ds(k * SEGW + i * L, L)]
                gid = o + i * L + lane
                Vw, Pw = _merge5(Vw, Pw, x, gid)
                return tuple(Vw) + tuple(Pw)
            kc = lax.fori_loop(0, SEGW // L, win_body, tuple(V) + tuple(P))
            V = list(kc[0:5])
            P = list(kc[5:10])
        V, P = _merge5(V, P, aux_vm[pl.ds(0, L)], BODY + lane)
        V, P = _merge5(V, P, aux_vm[pl.ds(L, L)], BODY + L + lane)

        cv = minf
        ci = zero_i
        for k in range(5):
            v, p = _argmax5(V, P)
            for j in range(5):
                V[j] = jnp.where(P[j] == p, NEG_INF, V[j])
            cv = jnp.where(lane == k, v, cv)
            ci = jnp.where(lane == k, p, ci)

        meta = aux_vm[pl.ds(2 * L, L)]
        llp_r = jnp.max(jnp.where(lane == 0, meta, NEG_INF))
        pred_r = jnp.max(jnp.where(lane == 1, plsc.bitcast(meta, jnp.int32),
                                   jnp.int32(-1)))
        ended = pred_r == END
        end_vals = jnp.where(lane == 0, jnp.float32(0.0), minf)
        end_idx = jnp.where(lane == 0, 2,
                            jnp.where(lane == 1, 0,
                                      jnp.where(lane == 2, 1, lane)))
        cv = jnp.where(ended, end_vals, cv) + llp_r
        ci = jnp.where(ended, end_idx, ci)
        cand_vals[pl.ds(r * L, L)] = cv
        cand_idx[pl.ds(r * L, L)] = ci
        return 0

    lax.fori_loop(0, BEAM, row_body, 0)

    V = [jnp.full((L,), NEG_INF, jnp.float32) for _ in range(5)]
    P = [jnp.zeros((L,), jnp.int32) for _ in range(5)]
    for r in range(BEAM):
        x = cand_vals[pl.ds(r * L, L)]
        pos = r * L + lane
        V, P = _merge5(V, P, x, pos)

    ov = jnp.full((L,), NEG_INF, jnp.float32)
    oc = jnp.zeros((L,), jnp.int32)
    ob = jnp.zeros((L,), jnp.int32)
    for k in range(5):
        v, p = _argmax5(V, P)
        for j in range(5):
            V[j] = jnp.where(P[j] == p, NEG_INF, V[j])
        cls_vec = plsc.load_gather(cand_idx, [jnp.full((L,), p, jnp.int32)])
        ov = jnp.where(lane == k, v, ov)
        oc = jnp.where(lane == k, cls_vec, oc)
        ob = jnp.where(lane == k, p >> 4, ob)

    o_all[pl.ds(0, L)] = oc
    o_all[pl.ds(L, L)] = plsc.bitcast(ov, jnp.int32)
    o_all[pl.ds(2 * L, L)] = ob
    pltpu.make_async_copy(o_all, out_hbm.at[w], osem).start()
    pltpu.make_async_copy(o_all, out_hbm.at[w], osem).wait()


@jax.jit
def kernel(class_log_probs, last_log_probs, last_predictions):
    llp_flat = last_log_probs.reshape(-1)
    lp_bits = lax.bitcast_convert_type(last_predictions.astype(jnp.int32),
                                       jnp.float32)
    aux = jnp.concatenate(
        [class_log_probs[:, BODY:], llp_flat[:, None], lp_bits[:, None],
         jnp.zeros((BATCH * BEAM, L - 2), jnp.float32)], axis=1)

    mesh = plsc.VectorSubcoreMesh(core_axis_name="c", subcore_axis_name="s",
                                  num_cores=NC, num_subcores=NS)
    sc_call = pl.kernel(
        _sc_body,
        out_type=[
            jax.ShapeDtypeStruct((BATCH, 3 * L), jnp.int32),
        ],
        mesh=mesh,
        compiler_params=pltpu.CompilerParams(needs_layout_passes=False),
        scratch_types=[
            pltpu.VMEM((1, CHUNK), jnp.float32),
            pltpu.VMEM((1, CHUNK), jnp.float32),
            pltpu.VMEM((1, CHUNK), jnp.float32),
            pltpu.VMEM((1, 5 * SEGW), jnp.float32),
            pltpu.VMEM((3 * L,), jnp.float32),
            pltpu.VMEM((L,), jnp.int32),
            pltpu.VMEM((BEAM * L,), jnp.float32),
            pltpu.VMEM((BEAM * L,), jnp.int32),
            pltpu.VMEM((3 * L,), jnp.int32),
            pltpu.SemaphoreType.DMA,
            pltpu.SemaphoreType.DMA,
            pltpu.SemaphoreType.DMA,
            pltpu.SemaphoreType.DMA,
            pltpu.SemaphoreType.DMA,
        ],
    )
    [out] = sc_call(class_log_probs, aux)
    preds = out[:, :BEAM]
    vals = lax.bitcast_convert_type(out[:, L:L + BEAM], jnp.float32)
    bp = out[:, 2 * L:2 * L + BEAM]
    return (preds, vals, bp)


# speedup vs baseline: 1.1151x; 1.0035x over previous
import jax
import jax.numpy as jnp
from jax import lax
from jax.experimental import pallas as pl
from jax.experimental.pallas import tpu as pltpu
from jax.experimental.pallas import tpu_sc as plsc

END = 2
BATCH = 32
BEAM = 5
VOCAB = 100000
L = 16
NC, NS = 2, 16
BODY = 99968
SEGW = 512
CHUNK = 19968
NCHUNK = 5
SEGS_PER_CHUNK = CHUNK // SEGW
SHORT_OFF = NCHUNK * CHUNK
NSEG = 196
LAST_WIN = BODY - SEGW
NEG_INF = float("-inf")
BIG = 2**31 - 1

FETCH_PLAN = tuple((t * CHUNK, CHUNK) for t in range(NCHUNK)) + ((SHORT_OFF, 128),)


def _merge5(V, P, x, pos):
    for j in range(5):
        m = x > V[j]
        nv = jnp.where(m, x, V[j])
        np_ = jnp.where(m, pos, P[j])
        x = jnp.where(m, V[j], x)
        pos = jnp.where(m, P[j], pos)
        V[j], P[j] = nv, np_
    return V, P


def _argmax5(V, P):
    rv, rp = V[0], P[0]
    for j in range(1, 5):
        m = V[j] > rv
        rv = jnp.where(m, V[j], rv)
        rp = jnp.where(m, P[j], rp)
    v = jnp.max(rv)
    p = jnp.min(jnp.where(rv == v, rp, BIG))
    return v, p


def _sc_body(clp_hbm, aux_hbm,
             out_hbm,
             buf0, buf1, buf2, union_v, aux_vm, idx_vm,
             cand_vals, cand_idx, o_all,
             sem0, sem1, sem2, gsem, osem):
    w = lax.axis_index("s") * NC + lax.axis_index("c")
    lane = lax.iota(jnp.int32, L)
    minf = jnp.full((L,), NEG_INF, jnp.float32)
    zero_i = jnp.zeros((L,), jnp.int32)

    bufs = (buf0, buf1, buf2)
    sems = (sem0, sem1, sem2)

    def fetch(t, idx_slot):
        off, width = FETCH_PLAN[t]
        return pltpu.make_async_copy(
            clp_hbm.at[idx_slot, pl.ds(off, width)],
            bufs[t % 3].at[pl.ds(0, 1), pl.ds(0, width)],
            sems[t % 3])


    g0 = w * BEAM
    idx_vm[...] = jnp.full((L,), g0, jnp.int32)
    for t in range(3):
        fetch(t, idx_vm.at[pl.ds(0, 1)]).start()

    def row_body(r, _):
        g = w * BEAM + r
        cur_off = (r % 2) * 8
        nxt_off = 8 - cur_off
        idx_cur = idx_vm.at[pl.ds(cur_off, 1)]
        idx_nxt = idx_vm.at[pl.ds(nxt_off, 1)]
        idx_vm[...] = jnp.where(lane == nxt_off, g + 1, g)
        pltpu.sync_copy(aux_hbm.at[g], aux_vm)

        carry = (minf,) * 5 + (zero_i,) * 5
        for t in range(NCHUNK):
            fetch(t, idx_cur).wait()
            cur = bufs[t % 3]

            def seg_body(s, kc, cur=cur, base=t * SEGS_PER_CHUNK):
                V = list(kc[0:5])
                P = list(kc[5:10])
                acc = cur[0, pl.ds(s * SEGW, L)]
                for j in range(1, SEGW // L):
                    acc = jnp.maximum(acc, cur[0, pl.ds(s * SEGW + j * L, L)])
                pos = (base + s) * L + lane
                V, P = _merge5(V, P, acc, pos)
                return tuple(V) + tuple(P)
            carry = lax.fori_loop(0, SEGS_PER_CHUNK, seg_body, carry)

            if t + 3 <= NCHUNK:
                fetch(t + 3, idx_cur).start()
            else:
                @pl.when(r < BEAM - 1)
                def _():
                    fetch(t + 3 - 6, idx_nxt).start()

        fetch(5, idx_cur).wait()
        V = list(carry[0:5])
        P = list(carry[5:10])
        acc = bufs[2][0, pl.ds(0, L)]
        for j in range(1, 128 // L):
            acc = jnp.maximum(acc, bufs[2][0, pl.ds(j * L, L)])
        V, P = _merge5(V, P, acc, (NSEG - 1) * L + lane)

        @pl.when(r < BEAM - 1)
        def _():
            fetch(2, idx_nxt).start()

        found = []
        for _k in range(5):
            _, p = _argmax5(V, P)
            parent = p >> 4
            found.append(parent)
            for j in range(5):
                V[j] = jnp.where((P[j] >> 4) == parent, NEG_INF, V[j])

        offs = []
        for k in range(5):
            o = found[k] * SEGW
            o = jnp.where(o > LAST_WIN, LAST_WIN, o)
            offs.append(pl.multiple_of(o, 128))
        for k in range(5):
            pltpu.make_async_copy(
                clp_hbm.at[idx_cur, pl.ds(offs[k], SEGW)],
                union_v.at[pl.ds(0, 1), pl.ds(k * SEGW, SEGW)], gsem).start()
        for k in range(5):
            pltpu.make_async_copy(
                clp_hbm.at[idx_cur, pl.ds(offs[k], SEGW)],
                union_v.at[pl.ds(0, 1), pl.ds(k * SEGW, SEGW)], gsem).wait()

        V = [minf] * 5
        P = [zero_i] * 5
        for k in range(5):
            def win_body(i, kc, k=k, o=offs[k]):
                Vw = list(kc[0:5])
                Pw = list(kc[5:10])
                x = union_v[0, pl.ds(k * SEGW + i * L, L)]
                gid = o + i * L + lane
                Vw, Pw = _merge5(Vw, Pw, x, gid)
                return tuple(Vw) + tuple(Pw)
            kc = lax.fori_loop(0, SEGW // L, win_body, tuple(V) + tuple(P))
            V = list(kc[0:5])
            P = list(kc[5:10])
        V, P = _merge5(V, P, aux_vm[pl.ds(0, L)], BODY + lane)
        V, P = _merge5(V, P, aux_vm[pl.ds(L, L)], BODY + L + lane)

        cv = minf
        ci = zero_i
        for k in range(5):
            v, p = _argmax5(V, P)
            for j in range(5):
                V[j] = jnp.where(P[j] == p, NEG_INF, V[j])
            cv = jnp.where(lane == k, v, cv)
            ci = jnp.where(lane == k, p, ci)

        meta = aux_vm[pl.ds(2 * L, L)]
        llp_r = jnp.max(jnp.where(lane == 0, meta, NEG_INF))
        pred_r = jnp.max(jnp.where(lane == 1, meta, NEG_INF))
        ended = pred_r == jnp.float32(END)
        end_vals = jnp.where(lane == 0, jnp.float32(0.0), minf)
        end_idx = jnp.where(lane == 0, 2,
                            jnp.where(lane == 1, 0,
                                      jnp.where(lane == 2, 1, lane)))
        cv = jnp.where(ended, end_vals, cv) + llp_r
        ci = jnp.where(ended, end_idx, ci)
        cand_vals[pl.ds(r * L, L)] = cv
        cand_idx[pl.ds(r * L, L)] = ci
        return 0

    lax.fori_loop(0, BEAM, row_body, 0)

    V = [jnp.full((L,), NEG_INF, jnp.float32) for _ in range(5)]
    P = [jnp.zeros((L,), jnp.int32) for _ in range(5)]
    for r in range(BEAM):
        x = cand_vals[pl.ds(r * L, L)]
        pos = r * L + lane
        V, P = _merge5(V, P, x, pos)

    ov = jnp.full((L,), NEG_INF, jnp.float32)
    oc = jnp.zeros((L,), jnp.int32)
    ob = jnp.zeros((L,), jnp.int32)
    for k in range(5):
        v, p = _argmax5(V, P)
        for j in range(5):
            V[j] = jnp.where(P[j] == p, NEG_INF, V[j])
        cls_vec = plsc.load_gather(cand_idx, [jnp.full((L,), p, jnp.int32)])
        ov = jnp.where(lane == k, v, ov)
        oc = jnp.where(lane == k, cls_vec, oc)
        ob = jnp.where(lane == k, p >> 4, ob)

    o_all[pl.ds(0, L)] = oc
    o_all[pl.ds(L, L)] = plsc.bitcast(ov, jnp.int32)
    o_all[pl.ds(2 * L, L)] = ob
    pltpu.make_async_copy(o_all, out_hbm.at[w], osem).start()
    pltpu.make_async_copy(o_all, out_hbm.at[w], osem).wait()


@jax.jit
def kernel(class_log_probs, last_log_probs, last_predictions):
    llp_flat = last_log_probs.reshape(-1)
    lp_f = last_predictions.astype(jnp.float32)
    aux = jnp.concatenate(
        [class_log_probs[:, BODY:], llp_flat[:, None], lp_f[:, None],
         jnp.zeros((BATCH * BEAM, L - 2), jnp.float32)], axis=1)

    mesh = plsc.VectorSubcoreMesh(core_axis_name="c", subcore_axis_name="s",
                                  num_cores=NC, num_subcores=NS)
    sc_call = pl.kernel(
        _sc_body,
        out_type=[
            jax.ShapeDtypeStruct((BATCH, 3 * L), jnp.int32),
        ],
        mesh=mesh,
        compiler_params=pltpu.CompilerParams(needs_layout_passes=False),
        scratch_types=[
            pltpu.VMEM((1, CHUNK), jnp.float32),
            pltpu.VMEM((1, CHUNK), jnp.float32),
            pltpu.VMEM((1, CHUNK), jnp.float32),
            pltpu.VMEM((1, 5 * SEGW), jnp.float32),
            pltpu.VMEM((3 * L,), jnp.float32),
            pltpu.VMEM((L,), jnp.int32),
            pltpu.VMEM((BEAM * L,), jnp.float32),
            pltpu.VMEM((BEAM * L,), jnp.int32),
            pltpu.VMEM((3 * L,), jnp.int32),
            pltpu.SemaphoreType.DMA,
            pltpu.SemaphoreType.DMA,
            pltpu.SemaphoreType.DMA,
            pltpu.SemaphoreType.DMA,
            pltpu.SemaphoreType.DMA,
        ],
    )
    [out] = sc_call(class_log_probs, aux)
    preds = out[:, :BEAM]
    vals = lax.bitcast_convert_type(out[:, L:L + BEAM], jnp.float32)
    bp = out[:, 2 * L:2 * L + BEAM]
    return (preds, vals, bp)
